# baseline (speedup 1.0000x reference)
import functools
import math

import jax
import jax.numpy as jnp
from jax.experimental import pallas as pl
from jax.experimental.pallas import tpu as pltpu
from jax.experimental.pallas import tpu_sc as plsc

B = 1024
D = 128
N = 100000
TILE_N = 5000
CHUNKS = 5
TILE_C = TILE_N // CHUNKS
TEMP = 0.05
LOG2E = math.log2(math.e)
QSCALE = LOG2E / TEMP

_info = plsc.get_sparse_core_info()
_NW = _info.num_cores * _info.num_subcores
_BPW = B // _NW


def _gather_body(f_rgb_hbm, t_rgb_hbm, f_ir_hbm, t_ir_hbm,
                 g_rgb_hbm, g_ir_hbm, idx_v, rows_v, sem):
    wid = (jax.lax.axis_index("s") * _info.num_cores
           + jax.lax.axis_index("c"))
    base = wid * _BPW
    for f_hbm, t_hbm, g_hbm in ((f_rgb_hbm, t_rgb_hbm, g_rgb_hbm),
                                (f_ir_hbm, t_ir_hbm, g_ir_hbm)):
        pltpu.sync_copy(t_hbm.at[pl.ds(base, _BPW)], idx_v)
        pltpu.async_copy(f_hbm.at[idx_v], rows_v, sem).wait()
        pltpu.sync_copy(rows_v, g_hbm.at[pl.ds(base, _BPW)])


_sc_gather = pl.kernel(
    _gather_body,
    mesh=plsc.VectorSubcoreMesh(core_axis_name="c", subcore_axis_name="s"),
    out_type=[jax.ShapeDtypeStruct((B, D), jnp.float32)] * 2,
    scratch_types=[
        pltpu.VMEM((_BPW,), jnp.int32),
        pltpu.VMEM((_BPW, D), jnp.float32),
        pltpu.SemaphoreType.DMA,
    ],
)


def _cm_kernel(x_rgb_ref, x_ir_ref, g_rgb_ref, g_ir_ref,
               f_rgb_ref, f_ir_ref, out_rgb_ref, out_ir_ref,
               xn_rgb, xn_ir, inv_rgb, inv_ir, s_rgb, s_ir):
    c = pl.program_id(0)
    nc = pl.num_programs(0)

    @pl.when(c == 0)
    def _init():
        for x_ref, xn, inv in ((x_rgb_ref, xn_rgb, inv_rgb),
                               (x_ir_ref, xn_ir, inv_ir)):
            x = x_ref[...]
            n = jnp.sqrt(jnp.sum(x * x, axis=1, keepdims=True))
            r = 1.0 / jnp.maximum(n, 1e-12)
            xn[...] = (x * (QSCALE * r)).astype(jnp.bfloat16)
            inv[...] = r * (1.0 / TEMP)
        s_rgb[...] = jnp.zeros_like(s_rgb)
        s_ir[...] = jnp.zeros_like(s_ir)

    def bank(xn, f_ref, s_acc):
        acc = None
        for k in range(CHUNKS):
            q = jax.lax.dot_general(
                xn[...], f_ref[k * TILE_C:(k + 1) * TILE_C, :].astype(
                    jnp.bfloat16),
                (((1,), (1,)), ((), ())),
                preferred_element_type=jnp.float32)
            part = jnp.sum(jnp.exp2(q), axis=1, keepdims=True)
            acc = part if acc is None else acc + part
        s_acc[...] += acc

    bank(xn_rgb, f_rgb_ref, s_rgb)
    bank(xn_ir, f_ir_ref, s_ir)

    @pl.when(c == nc - 1)
    def _fin():
        for x_ref, g_ref, inv, s_acc, out_ref in (
                (x_rgb_ref, g_rgb_ref, inv_rgb, s_rgb, out_rgb_ref),
                (x_ir_ref, g_ir_ref, inv_ir, s_ir, out_ir_ref)):
            picked = jnp.sum(x_ref[...] * g_ref[...], axis=1,
                             keepdims=True) * inv[...]
            out_ref[...] = jnp.mean(
                jnp.log(s_acc[...]) - picked).reshape(1, 1)


@jax.jit
def _run(x_rgb, x_ir, t_rgb, t_ir, f_rgb, f_ir):
    g_rgb, g_ir = _sc_gather(f_rgb, t_rgb, f_ir, t_ir)
    out = pl.pallas_call(
        _cm_kernel,
        grid=(N // TILE_N,),
        in_specs=[
            pl.BlockSpec((B, D), lambda c: (0, 0)),
            pl.BlockSpec((B, D), lambda c: (0, 0)),
            pl.BlockSpec((B, D), lambda c: (0, 0)),
            pl.BlockSpec((B, D), lambda c: (0, 0)),
            pl.BlockSpec((TILE_N, D), lambda c: (c, 0)),
            pl.BlockSpec((TILE_N, D), lambda c: (c, 0)),
        ],
        out_specs=[
            pl.BlockSpec((1, 1), lambda c: (0, 0)),
            pl.BlockSpec((1, 1), lambda c: (0, 0)),
        ],
        out_shape=[jax.ShapeDtypeStruct((1, 1), jnp.float32)] * 2,
        scratch_shapes=[
            pltpu.VMEM((B, D), jnp.bfloat16),
            pltpu.VMEM((B, D), jnp.bfloat16),
            pltpu.VMEM((B, 1), jnp.float32),
            pltpu.VMEM((B, 1), jnp.float32),
            pltpu.VMEM((B, 1), jnp.float32),
            pltpu.VMEM((B, 1), jnp.float32),
        ],
        compiler_params=pltpu.CompilerParams(
            dimension_semantics=("arbitrary",)),
    )(x_rgb, x_ir, g_rgb, g_ir, f_rgb, f_ir)
    return out[0][0, 0], out[1][0, 0]


def kernel(inputs_rgb, inputs_ir, targets_rgb, targets_ir,
           features_rgb, features_ir):
    return _run(inputs_rgb, inputs_ir, targets_rgb, targets_ir,
                features_rgb, features_ir)


# speedup vs baseline: 1.0617x; 1.0617x over previous
import functools
import math

import jax
import jax.numpy as jnp
from jax.experimental import pallas as pl
from jax.experimental.pallas import tpu as pltpu
from jax.experimental.pallas import tpu_sc as plsc

B = 1024
D = 128
N = 100000
TILE_N = 5000
TEMP = 0.05
LOG2E = math.log2(math.e)
QSCALE = LOG2E / TEMP

_info = plsc.get_sparse_core_info()
_NW = _info.num_cores * _info.num_subcores
_BPW = B // _NW


def _gather_body(f_rgb_hbm, t_rgb_hbm, f_ir_hbm, t_ir_hbm,
                 g_rgb_hbm, g_ir_hbm, idx_v, rows_v, sem):
    wid = (jax.lax.axis_index("s") * _info.num_cores
           + jax.lax.axis_index("c"))
    base = wid * _BPW
    for f_hbm, t_hbm, g_hbm in ((f_rgb_hbm, t_rgb_hbm, g_rgb_hbm),
                                (f_ir_hbm, t_ir_hbm, g_ir_hbm)):
        pltpu.sync_copy(t_hbm.at[pl.ds(base, _BPW)], idx_v)
        pltpu.async_copy(f_hbm.at[idx_v], rows_v, sem).wait()
        pltpu.sync_copy(rows_v, g_hbm.at[pl.ds(base, _BPW)])


_sc_gather = pl.kernel(
    _gather_body,
    mesh=plsc.VectorSubcoreMesh(core_axis_name="c", subcore_axis_name="s"),
    out_type=[jax.ShapeDtypeStruct((B, D), jnp.float32)] * 2,
    scratch_types=[
        pltpu.VMEM((_BPW,), jnp.int32),
        pltpu.VMEM((_BPW, D), jnp.float32),
        pltpu.SemaphoreType.DMA,
    ],
)


def _sums_kernel(x_rgb_ref, x_ir_ref, f_rgb_ref, f_ir_ref,
                 s_rgb_ref, s_ir_ref, xn_rgb, xn_ir):
    c = pl.program_id(0)

    @pl.when(c == 0)
    def _init():
        for x_ref, xn in ((x_rgb_ref, xn_rgb), (x_ir_ref, xn_ir)):
            x = x_ref[...]
            n = jnp.sqrt(jnp.sum(x * x, axis=1, keepdims=True))
            xn[...] = (x * (QSCALE / jnp.maximum(n, 1e-12))).astype(
                jnp.bfloat16)
        s_rgb_ref[...] = jnp.zeros_like(s_rgb_ref)
        s_ir_ref[...] = jnp.zeros_like(s_ir_ref)

    def bank(xn, f_ref, s_ref):
        q = jax.lax.dot_general(
            xn[...], f_ref[...].astype(jnp.bfloat16),
            (((1,), (1,)), ((), ())),
            preferred_element_type=jnp.float32)
        s_ref[...] += jnp.sum(jnp.exp2(q), axis=1, keepdims=True)

    bank(xn_rgb, f_rgb_ref, s_rgb_ref)
    bank(xn_ir, f_ir_ref, s_ir_ref)


def _tail_kernel(x_rgb_ref, x_ir_ref, g_rgb_ref, g_ir_ref,
                 s_rgb_ref, s_ir_ref, out_rgb_ref, out_ir_ref):
    for x_ref, g_ref, s_ref, out_ref in (
            (x_rgb_ref, g_rgb_ref, s_rgb_ref, out_rgb_ref),
            (x_ir_ref, g_ir_ref, s_ir_ref, out_ir_ref)):
        x = x_ref[...]
        n = jnp.sqrt(jnp.sum(x * x, axis=1, keepdims=True))
        inv = (1.0 / TEMP) / jnp.maximum(n, 1e-12)
        picked = jnp.sum(x * g_ref[...], axis=1, keepdims=True) * inv
        out_ref[...] = jnp.mean(jnp.log(s_ref[...]) - picked).reshape(1, 1)


@jax.jit
def _run(x_rgb, x_ir, t_rgb, t_ir, f_rgb, f_ir):
    g_rgb, g_ir = _sc_gather(f_rgb, t_rgb, f_ir, t_ir)
    s_rgb, s_ir = pl.pallas_call(
        _sums_kernel,
        grid=(N // TILE_N,),
        in_specs=[
            pl.BlockSpec((B, D), lambda c: (0, 0)),
            pl.BlockSpec((B, D), lambda c: (0, 0)),
            pl.BlockSpec((TILE_N, D), lambda c: (c, 0)),
            pl.BlockSpec((TILE_N, D), lambda c: (c, 0)),
        ],
        out_specs=[
            pl.BlockSpec((B, 1), lambda c: (0, 0)),
            pl.BlockSpec((B, 1), lambda c: (0, 0)),
        ],
        out_shape=[jax.ShapeDtypeStruct((B, 1), jnp.float32)] * 2,
        scratch_shapes=[
            pltpu.VMEM((B, D), jnp.bfloat16),
            pltpu.VMEM((B, D), jnp.bfloat16),
        ],
        compiler_params=pltpu.CompilerParams(
            dimension_semantics=("arbitrary",)),
    )(x_rgb, x_ir, f_rgb, f_ir)
    out = pl.pallas_call(
        _tail_kernel,
        out_shape=[jax.ShapeDtypeStruct((1, 1), jnp.float32)] * 2,
    )(x_rgb, x_ir, g_rgb, g_ir, s_rgb, s_ir)
    return out[0][0, 0], out[1][0, 0]


def kernel(inputs_rgb, inputs_ir, targets_rgb, targets_ir,
           features_rgb, features_ir):
    return _run(inputs_rgb, inputs_ir, targets_rgb, targets_ir,
                features_rgb, features_ir)


# speedup vs baseline: 1.0643x; 1.0024x over previous
import functools
import math

import jax
import jax.numpy as jnp
from jax.experimental import pallas as pl
from jax.experimental.pallas import tpu as pltpu
from jax.experimental.pallas import tpu_sc as plsc

B = 1024
D = 128
N = 100000
TILE_N = 5000
TEMP = 0.05
LOG2E = math.log2(math.e)
QSCALE = LOG2E / TEMP

_info = plsc.get_sparse_core_info()
_NW = _info.num_cores * _info.num_subcores
_BPW = B // _NW


def _gather_body(f_rgb_hbm, t_rgb_hbm, f_ir_hbm, t_ir_hbm,
                 g_rgb_hbm, g_ir_hbm, idx_a, idx_b, rows_a, rows_b,
                 sem_a, sem_b):
    wid = (jax.lax.axis_index("s") * _info.num_cores
           + jax.lax.axis_index("c"))
    base = wid * _BPW
    pltpu.sync_copy(t_rgb_hbm.at[pl.ds(base, _BPW)], idx_a)
    pltpu.sync_copy(t_ir_hbm.at[pl.ds(base, _BPW)], idx_b)
    cp_a = pltpu.async_copy(f_rgb_hbm.at[idx_a], rows_a, sem_a)
    cp_b = pltpu.async_copy(f_ir_hbm.at[idx_b], rows_b, sem_b)
    cp_a.wait()
    pltpu.sync_copy(rows_a, g_rgb_hbm.at[pl.ds(base, _BPW)])
    cp_b.wait()
    pltpu.sync_copy(rows_b, g_ir_hbm.at[pl.ds(base, _BPW)])


_sc_gather = pl.kernel(
    _gather_body,
    mesh=plsc.VectorSubcoreMesh(core_axis_name="c", subcore_axis_name="s"),
    out_type=[jax.ShapeDtypeStruct((B, D), jnp.float32)] * 2,
    scratch_types=[
        pltpu.VMEM((_BPW,), jnp.int32),
        pltpu.VMEM((_BPW,), jnp.int32),
        pltpu.VMEM((_BPW, D), jnp.float32),
        pltpu.VMEM((_BPW, D), jnp.float32),
        pltpu.SemaphoreType.DMA,
        pltpu.SemaphoreType.DMA,
    ],
)


def _sums_kernel(x_rgb_ref, x_ir_ref, f_rgb_ref, f_ir_ref,
                 s_rgb_ref, s_ir_ref, xn_rgb, xn_ir):
    c = pl.program_id(0)

    @pl.when(c == 0)
    def _init():
        for x_ref, xn in ((x_rgb_ref, xn_rgb), (x_ir_ref, xn_ir)):
            x = x_ref[...]
            n = jnp.sqrt(jnp.sum(x * x, axis=1, keepdims=True))
            xn[...] = (x * (QSCALE / jnp.maximum(n, 1e-12))).astype(
                jnp.bfloat16)
        s_rgb_ref[...] = jnp.zeros_like(s_rgb_ref)
        s_ir_ref[...] = jnp.zeros_like(s_ir_ref)

    def bank(xn, f_ref, s_ref):
        q = jax.lax.dot_general(
            xn[...], f_ref[...].astype(jnp.bfloat16),
            (((1,), (1,)), ((), ())),
            preferred_element_type=jnp.float32)
        s_ref[...] += jnp.sum(jnp.exp2(q), axis=1, keepdims=True)

    bank(xn_rgb, f_rgb_ref, s_rgb_ref)
    bank(xn_ir, f_ir_ref, s_ir_ref)


def _tail_kernel(x_rgb_ref, x_ir_ref, g_rgb_ref, g_ir_ref,
                 s_rgb_ref, s_ir_ref, out_rgb_ref, out_ir_ref):
    for x_ref, g_ref, s_ref, out_ref in (
            (x_rgb_ref, g_rgb_ref, s_rgb_ref, out_rgb_ref),
            (x_ir_ref, g_ir_ref, s_ir_ref, out_ir_ref)):
        x = x_ref[...]
        n = jnp.sqrt(jnp.sum(x * x, axis=1, keepdims=True))
        inv = (1.0 / TEMP) / jnp.maximum(n, 1e-12)
        picked = jnp.sum(x * g_ref[...], axis=1, keepdims=True) * inv
        out_ref[...] = jnp.mean(jnp.log(s_ref[...]) - picked).reshape(1, 1)


@jax.jit
def _run(x_rgb, x_ir, t_rgb, t_ir, f_rgb, f_ir):
    g_rgb, g_ir = _sc_gather(f_rgb, t_rgb, f_ir, t_ir)
    s_rgb, s_ir = pl.pallas_call(
        _sums_kernel,
        grid=(N // TILE_N,),
        in_specs=[
            pl.BlockSpec((B, D), lambda c: (0, 0)),
            pl.BlockSpec((B, D), lambda c: (0, 0)),
            pl.BlockSpec((TILE_N, D), lambda c: (c, 0)),
            pl.BlockSpec((TILE_N, D), lambda c: (c, 0)),
        ],
        out_specs=[
            pl.BlockSpec((B, 1), lambda c: (0, 0)),
            pl.BlockSpec((B, 1), lambda c: (0, 0)),
        ],
        out_shape=[jax.ShapeDtypeStruct((B, 1), jnp.float32)] * 2,
        scratch_shapes=[
            pltpu.VMEM((B, D), jnp.bfloat16),
            pltpu.VMEM((B, D), jnp.bfloat16),
        ],
        compiler_params=pltpu.CompilerParams(
            dimension_semantics=("arbitrary",)),
    )(x_rgb, x_ir, f_rgb, f_ir)
    out = pl.pallas_call(
        _tail_kernel,
        out_shape=[jax.ShapeDtypeStruct((1, 1), jnp.float32)] * 2,
    )(x_rgb, x_ir, g_rgb, g_ir, s_rgb, s_ir)
    return out[0][0, 0], out[1][0, 0]


def kernel(inputs_rgb, inputs_ir, targets_rgb, targets_ir,
           features_rgb, features_ir):
    return _run(inputs_rgb, inputs_ir, targets_rgb, targets_ir,
                features_rgb, features_ir)
